# Initial kernel scaffold; baseline (speedup 1.0000x reference)
import jax
import jax.numpy as jnp
from jax.experimental import pallas as pl


def kernel(x, edge_index, mask, adj, emb, params):
    raise NotImplementedError("write your pallas kernel here")


# speedup vs baseline: 1.8627x; 1.8627x over previous
import jax
import jax.numpy as jnp
from jax.experimental import pallas as pl
from jax.experimental.pallas import tpu as pltpu

_B, _S, _N = 4, 16, 512
_DH = 32


def _mm(a, b):
    return jax.lax.dot_general(a, b, (((1,), (0,)), ((), ())),
                               preferred_element_type=jnp.float32)


def _grinet_body(xs_ref, ms_ref, adj_ref, adjT_ref, embr_ref,
                 fWd, fbd, fWrz, fbrz, fWc, fbc, fWro, fbro, fWro2, fbro2,
                 bWd, bbd, bWrz, bbrz, bWc, bbc, bWro, bbro, bWro2, bbro2,
                 Wm1, bm1, Wm2, bm2,
                 out_ref,
                 fimp, bimp, fh, bh):
    adj = adj_ref[...]
    adjT = adjT_ref[...]
    s1 = adj / jnp.clip(jnp.sum(adj, axis=1, keepdims=True), 1e-8, None)
    s2 = adjT / jnp.clip(jnp.sum(adjT, axis=1, keepdims=True), 1e-8, None)
    s12 = jnp.concatenate([s1, s2], axis=0)

    def run_dir(Wd, bd, Wrz, brz, Wc, bc, Wro, bro, Wro2, bro2,
                imp_ref, h_ref, reverse):
        Wd_ = Wd[...]; bd_ = bd[...]
        Wrz_ = Wrz[...]; brz_ = brz[...]
        Wc_ = Wc[...]; bc_ = bc[...]
        Wro_ = Wro[...]; bro_ = bro[...]
        Wro2_ = Wro2[...]; bro2_ = bro2[...]

        def step(i, h):
            t = (_S - 1 - i) if reverse else i
            xs = xs_ref[t]
            ms = ms_ref[t]
            om = 1.0 - ms
            x1 = _mm(h, Wro_) + bro_
            xf1 = ms * xs + om * x1
            Xd = jnp.concatenate([xf1, ms, h], axis=1)
            g = _mm(s12, Xd)
            Gd = jnp.concatenate([Xd, g[:_N], g[_N:]], axis=1)
            dh = jnp.maximum(_mm(Gd, Wd_) + bd_, 0.0)
            x2 = _mm(dh, Wro2_) + bro2_
            xf2 = ms * xs + om * x2
            Xg = jnp.concatenate([xf2, ms, h], axis=1)
            a = _mm(s12, Xg)
            a1 = a[:_N]
            a2 = a[_N:]
            a11 = _mm(s1, a1)
            a22 = _mm(s2, a2)
            G = jnp.concatenate([Xg, a1, a11, a2, a22], axis=1)
            rz = jax.nn.sigmoid(_mm(G, Wrz_) + brz_)
            r = rz[:, :_DH * _B]
            z = rz[:, _DH * _B:]
            Xc = jnp.concatenate([xf2, ms, r * h], axis=1)
            ca = _mm(s12, Xc)
            c1 = ca[:_N]
            c2 = ca[_N:]
            c11 = _mm(s1, c1)
            c22 = _mm(s2, c2)
            Gc = jnp.concatenate([Xc, c1, c11, c2, c22], axis=1)
            c = jnp.tanh(_mm(Gc, Wc_) + bc_)
            hn = z * h + (1.0 - z) * c
            imp_ref[t] = x2
            h_ref[t] = hn
            return hn

        h0 = jnp.zeros((_N, _DH * _B), jnp.float32)
        jax.lax.fori_loop(0, _S, step, h0)

    run_dir(fWd, fbd, fWrz, fbrz, fWc, fbc, fWro, fbro, fWro2, fbro2,
            fimp, fh, False)
    run_dir(bWd, bbd, bWrz, bbrz, bWc, bbc, bWro, bbro, bWro2, bbro2,
            bimp, bh, True)

    embr = embr_ref[...]
    Wm1_ = Wm1[...]; bm1_ = bm1[...]
    Wm2_ = Wm2[...]; bm2_ = bm2[...]

    def mlp(t, carry):
        ms = ms_ref[t]
        xs = xs_ref[t]
        mi = jnp.concatenate([fimp[t], bimp[t], fh[t], bh[t], ms, embr],
                             axis=1)
        hmid = jnp.maximum(_mm(mi, Wm1_) + bm1_, 0.0)
        o = _mm(hmid, Wm2_) + bm2_
        out_ref[t] = jnp.where(ms > 0.5, xs, o)
        return carry

    jax.lax.fori_loop(0, _S, mlp, 0)


def _kron(W):
    return jnp.kron(W, jnp.eye(_B, dtype=W.dtype))


def _rep(b):
    return jnp.repeat(b, _B)[None, :]


@jax.jit
def kernel(x, edge_index, mask, adj, emb, params):
    del edge_index
    xs = jnp.transpose(x[..., 0], (1, 2, 0))
    ms = jnp.transpose(mask[..., 0].astype(jnp.float32), (1, 2, 0))
    adjT = adj.T
    embr = jnp.repeat(emb, _B, axis=1)

    def dirw(p):
        return [
            _kron(p['Wd']), _rep(p['bd']),
            _kron(jnp.concatenate([p['Wr'], p['Wz']], axis=1)),
            _rep(jnp.concatenate([p['br'], p['bz']])),
            _kron(p['Wc']), _rep(p['bc']),
            _kron(p['Wro']), _rep(p['bro']),
            _kron(p['Wro2']), _rep(p['bro2']),
        ]

    fw = dirw(params['fwd'])
    bw = dirw(params['bwd'])
    Wm1 = _kron(params['Wm1'])
    bm1 = _rep(params['bm1'])
    Wm2 = _kron(params['Wm2'])
    bm2 = _rep(params['bm2'])

    out = pl.pallas_call(
        _grinet_body,
        out_shape=jax.ShapeDtypeStruct((_S, _N, _B), jnp.float32),
        scratch_shapes=[
            pltpu.VMEM((_S, _N, _B), jnp.float32),
            pltpu.VMEM((_S, _N, _B), jnp.float32),
            pltpu.VMEM((_S, _N, _DH * _B), jnp.float32),
            pltpu.VMEM((_S, _N, _DH * _B), jnp.float32),
        ],
    )(xs, ms, adj, adjT, embr, *fw, *bw, Wm1, bm1, Wm2, bm2)

    return jnp.transpose(out, (2, 0, 1))[..., None]


# speedup vs baseline: 2.1877x; 1.1745x over previous
import jax
import jax.numpy as jnp
from jax.experimental import pallas as pl
from jax.experimental.pallas import tpu as pltpu

_B, _S, _N = 4, 16, 512
_DH = 32
_CB = 34 * _B


def _mm(a, b):
    return jax.lax.dot_general(a, b, (((1,), (0,)), ((), ())),
                               preferred_element_type=jnp.float32)


def _grinet_body(xs_ref, ms_ref, adj_ref, adjT_ref, embr_ref,
                 fWd, fbd, fWrz, fbrz, fWc, fbc, fWro, fbro, fWro2, fbro2,
                 bWd, bbd, bWrz, bbrz, bWc, bbc, bWro, bbro, bWro2, bbro2,
                 Wm1, bm1, Wm2, bm2,
                 out_ref,
                 fimp, bimp, fh, bh):
    adj = adj_ref[...]
    adjT = adjT_ref[...]
    s1 = adj / jnp.clip(jnp.sum(adj, axis=1, keepdims=True), 1e-8, None)
    s2 = adjT / jnp.clip(jnp.sum(adjT, axis=1, keepdims=True), 1e-8, None)
    s12 = jnp.concatenate([s1, s2], axis=0)

    fWd_ = fWd[...]; fbd_ = fbd[...]; bWd_ = bWd[...]; bbd_ = bbd[...]
    fWrz_ = fWrz[...]; fbrz_ = fbrz[...]; bWrz_ = bWrz[...]; bbrz_ = bbrz[...]
    fWc_ = fWc[...]; fbc_ = fbc[...]; bWc_ = bWc[...]; bbc_ = bbc[...]
    fWro_ = fWro[...]; fbro_ = fbro[...]; bWro_ = bWro[...]; bbro_ = bbro[...]
    fWro2_ = fWro2[...]; fbro2_ = fbro2[...]
    bWro2_ = bWro2[...]; bbro2_ = bbro2[...]

    def split(m):
        return m[:, :_CB], m[:, _CB:]

    def step(i, carry):
        hf, hb = carry
        tf = i
        tb = _S - 1 - i
        xsf = xs_ref[tf]; msf = ms_ref[tf]
        xsb = xs_ref[tb]; msb = ms_ref[tb]
        omf = 1.0 - msf
        omb = 1.0 - msb
        x1f = _mm(hf, fWro_) + fbro_
        x1b = _mm(hb, bWro_) + bbro_
        xf1f = msf * xsf + omf * x1f
        xf1b = msb * xsb + omb * x1b
        Xd = jnp.concatenate([xf1f, msf, hf, xf1b, msb, hb], axis=1)
        g = _mm(s12, Xd)
        g1f, g1b = split(g[:_N])
        g2f, g2b = split(g[_N:])
        Xdf, Xdb = split(Xd)
        dhf = jnp.maximum(
            _mm(jnp.concatenate([Xdf, g1f, g2f], axis=1), fWd_) + fbd_, 0.0)
        dhb = jnp.maximum(
            _mm(jnp.concatenate([Xdb, g1b, g2b], axis=1), bWd_) + bbd_, 0.0)
        x2f = _mm(dhf, fWro2_) + fbro2_
        x2b = _mm(dhb, bWro2_) + bbro2_
        xf2f = msf * xsf + omf * x2f
        xf2b = msb * xsb + omb * x2b
        Xg = jnp.concatenate([xf2f, msf, hf, xf2b, msb, hb], axis=1)
        a = _mm(s12, Xg)
        a1 = a[:_N]
        a2 = a[_N:]
        a11 = _mm(s1, a1)
        a22 = _mm(s2, a2)
        a1f, a1b = split(a1)
        a2f, a2b = split(a2)
        a11f, a11b = split(a11)
        a22f, a22b = split(a22)
        Xgf, Xgb = split(Xg)
        rzf = jax.nn.sigmoid(
            _mm(jnp.concatenate([Xgf, a1f, a11f, a2f, a22f], axis=1), fWrz_)
            + fbrz_)
        rzb = jax.nn.sigmoid(
            _mm(jnp.concatenate([Xgb, a1b, a11b, a2b, a22b], axis=1), bWrz_)
            + bbrz_)
        rf = rzf[:, :_DH * _B]; zf = rzf[:, _DH * _B:]
        rb = rzb[:, :_DH * _B]; zb = rzb[:, _DH * _B:]
        Xc = jnp.concatenate([xf2f, msf, rf * hf, xf2b, msb, rb * hb], axis=1)
        ca = _mm(s12, Xc)
        c1 = ca[:_N]
        c2 = ca[_N:]
        c11 = _mm(s1, c1)
        c22 = _mm(s2, c2)
        c1f, c1b = split(c1)
        c2f, c2b = split(c2)
        c11f, c11b = split(c11)
        c22f, c22b = split(c22)
        Xcf, Xcb = split(Xc)
        cf = jnp.tanh(
            _mm(jnp.concatenate([Xcf, c1f, c11f, c2f, c22f], axis=1), fWc_)
            + fbc_)
        cb = jnp.tanh(
            _mm(jnp.concatenate([Xcb, c1b, c11b, c2b, c22b], axis=1), bWc_)
            + bbc_)
        hfn = zf * hf + (1.0 - zf) * cf
        hbn = zb * hb + (1.0 - zb) * cb
        fimp[tf] = x2f
        fh[tf] = hfn
        bimp[tb] = x2b
        bh[tb] = hbn
        return (hfn, hbn)

    h0 = jnp.zeros((_N, _DH * _B), jnp.float32)
    jax.lax.fori_loop(0, _S, step, (h0, h0))

    SN = _S * _N
    embr = jnp.broadcast_to(embr_ref[...][None], (_S, _N, 8 * _B))
    mi = jnp.concatenate([
        fimp[...].reshape(SN, _B),
        bimp[...].reshape(SN, _B),
        fh[...].reshape(SN, _DH * _B),
        bh[...].reshape(SN, _DH * _B),
        ms_ref[...].reshape(SN, _B),
        embr.reshape(SN, 8 * _B),
    ], axis=1)
    hmid = jnp.maximum(_mm(mi, Wm1[...]) + bm1[...], 0.0)
    o = _mm(hmid, Wm2[...]) + bm2[...]
    msa = ms_ref[...].reshape(SN, _B)
    xsa = xs_ref[...].reshape(SN, _B)
    out_ref[...] = jnp.where(msa > 0.5, xsa, o).reshape(_S, _N, _B)


def _kron(W):
    return jnp.kron(W, jnp.eye(_B, dtype=W.dtype))


def _rep(b):
    return jnp.repeat(b, _B)[None, :]


@jax.jit
def kernel(x, edge_index, mask, adj, emb, params):
    del edge_index
    xs = jnp.transpose(x[..., 0], (1, 2, 0))
    ms = jnp.transpose(mask[..., 0].astype(jnp.float32), (1, 2, 0))
    adjT = adj.T
    embr = jnp.repeat(emb, _B, axis=1)

    def dirw(p):
        return [
            _kron(p['Wd']), _rep(p['bd']),
            _kron(jnp.concatenate([p['Wr'], p['Wz']], axis=1)),
            _rep(jnp.concatenate([p['br'], p['bz']])),
            _kron(p['Wc']), _rep(p['bc']),
            _kron(p['Wro']), _rep(p['bro']),
            _kron(p['Wro2']), _rep(p['bro2']),
        ]

    fw = dirw(params['fwd'])
    bw = dirw(params['bwd'])
    Wm1 = _kron(params['Wm1'])
    bm1 = _rep(params['bm1'])
    Wm2 = _kron(params['Wm2'])
    bm2 = _rep(params['bm2'])

    out = pl.pallas_call(
        _grinet_body,
        out_shape=jax.ShapeDtypeStruct((_S, _N, _B), jnp.float32),
        scratch_shapes=[
            pltpu.VMEM((_S, _N, _B), jnp.float32),
            pltpu.VMEM((_S, _N, _B), jnp.float32),
            pltpu.VMEM((_S, _N, _DH * _B), jnp.float32),
            pltpu.VMEM((_S, _N, _DH * _B), jnp.float32),
        ],
    )(xs, ms, adj, adjT, embr, *fw, *bw, Wm1, bm1, Wm2, bm2)

    return jnp.transpose(out, (2, 0, 1))[..., None]


# speedup vs baseline: 2.2015x; 1.0063x over previous
import jax
import jax.numpy as jnp
from jax.experimental import pallas as pl
from jax.experimental.pallas import tpu as pltpu

_B, _S, _N = 4, 16, 512
_DH = 32
_CB = 34 * _B


def _mm(a, b):
    return jax.lax.dot_general(a, b, (((1,), (0,)), ((), ())),
                               preferred_element_type=jnp.float32)


def _grinet_body(xs_ref, ms_ref, adj_ref, adjT_ref, embr_ref,
                 fWd, fbd, fWrz, fbrz, fWc, fbc, fWro, fbro, fWro2, fbro2,
                 bWd, bbd, bWrz, bbrz, bWc, bbc, bWro, bbro, bWro2, bbro2,
                 Wm1, bm1, Wm2, bm2,
                 out_ref,
                 fimp, bimp, fh, bh):
    adj = adj_ref[...]
    adjT = adjT_ref[...]
    s1 = adj / jnp.clip(jnp.sum(adj, axis=1, keepdims=True), 1e-8, None)
    s2 = adjT / jnp.clip(jnp.sum(adjT, axis=1, keepdims=True), 1e-8, None)
    s12 = jnp.concatenate([s1, s2], axis=0)
    s4 = jnp.concatenate([s12, _mm(s1, s1), _mm(s2, s2)], axis=0)

    fWd_ = fWd[...]; fbd_ = fbd[...]; bWd_ = bWd[...]; bbd_ = bbd[...]
    fWrz_ = fWrz[...]; fbrz_ = fbrz[...]; bWrz_ = bWrz[...]; bbrz_ = bbrz[...]
    fWc_ = fWc[...]; fbc_ = fbc[...]; bWc_ = bWc[...]; bbc_ = bbc[...]
    fWro_ = fWro[...]; fbro_ = fbro[...]; bWro_ = bWro[...]; bbro_ = bbro[...]
    fWro2_ = fWro2[...]; fbro2_ = fbro2[...]
    bWro2_ = bWro2[...]; bbro2_ = bbro2[...]

    def split(m):
        return m[:, :_CB], m[:, _CB:]

    def step(i, carry):
        hf, hb = carry
        tf = i
        tb = _S - 1 - i
        xsf = xs_ref[tf]; msf = ms_ref[tf]
        xsb = xs_ref[tb]; msb = ms_ref[tb]
        omf = 1.0 - msf
        omb = 1.0 - msb
        x1f = _mm(hf, fWro_) + fbro_
        x1b = _mm(hb, bWro_) + bbro_
        xf1f = msf * xsf + omf * x1f
        xf1b = msb * xsb + omb * x1b
        Xd = jnp.concatenate([xf1f, msf, hf, xf1b, msb, hb], axis=1)
        g = _mm(s12, Xd)
        g1f, g1b = split(g[:_N])
        g2f, g2b = split(g[_N:])
        Xdf, Xdb = split(Xd)
        dhf = jnp.maximum(
            _mm(jnp.concatenate([Xdf, g1f, g2f], axis=1), fWd_) + fbd_, 0.0)
        dhb = jnp.maximum(
            _mm(jnp.concatenate([Xdb, g1b, g2b], axis=1), bWd_) + bbd_, 0.0)
        x2f = _mm(dhf, fWro2_) + fbro2_
        x2b = _mm(dhb, bWro2_) + bbro2_
        xf2f = msf * xsf + omf * x2f
        xf2b = msb * xsb + omb * x2b
        Xg = jnp.concatenate([xf2f, msf, hf, xf2b, msb, hb], axis=1)
        a = _mm(s4, Xg)
        a1f, a1b = split(a[:_N])
        a2f, a2b = split(a[_N:2 * _N])
        a11f, a11b = split(a[2 * _N:3 * _N])
        a22f, a22b = split(a[3 * _N:])
        Xgf, Xgb = split(Xg)
        rzf = jax.nn.sigmoid(
            _mm(jnp.concatenate([Xgf, a1f, a11f, a2f, a22f], axis=1), fWrz_)
            + fbrz_)
        rzb = jax.nn.sigmoid(
            _mm(jnp.concatenate([Xgb, a1b, a11b, a2b, a22b], axis=1), bWrz_)
            + bbrz_)
        rf = rzf[:, :_DH * _B]; zf = rzf[:, _DH * _B:]
        rb = rzb[:, :_DH * _B]; zb = rzb[:, _DH * _B:]
        Xc = jnp.concatenate([xf2f, msf, rf * hf, xf2b, msb, rb * hb], axis=1)
        ca = _mm(s4, Xc)
        c1f, c1b = split(ca[:_N])
        c2f, c2b = split(ca[_N:2 * _N])
        c11f, c11b = split(ca[2 * _N:3 * _N])
        c22f, c22b = split(ca[3 * _N:])
        Xcf, Xcb = split(Xc)
        cf = jnp.tanh(
            _mm(jnp.concatenate([Xcf, c1f, c11f, c2f, c22f], axis=1), fWc_)
            + fbc_)
        cb = jnp.tanh(
            _mm(jnp.concatenate([Xcb, c1b, c11b, c2b, c22b], axis=1), bWc_)
            + bbc_)
        hfn = zf * hf + (1.0 - zf) * cf
        hbn = zb * hb + (1.0 - zb) * cb
        fimp[tf] = x2f
        fh[tf] = hfn
        bimp[tb] = x2b
        bh[tb] = hbn
        return (hfn, hbn)

    h0 = jnp.zeros((_N, _DH * _B), jnp.float32)
    jax.lax.fori_loop(0, _S, step, (h0, h0))

    SN = _S * _N
    embr = jnp.broadcast_to(embr_ref[...][None], (_S, _N, 8 * _B))
    mi = jnp.concatenate([
        fimp[...].reshape(SN, _B),
        bimp[...].reshape(SN, _B),
        fh[...].reshape(SN, _DH * _B),
        bh[...].reshape(SN, _DH * _B),
        ms_ref[...].reshape(SN, _B),
        embr.reshape(SN, 8 * _B),
    ], axis=1)
    hmid = jnp.maximum(_mm(mi, Wm1[...]) + bm1[...], 0.0)
    o = _mm(hmid, Wm2[...]) + bm2[...]
    msa = ms_ref[...].reshape(SN, _B)
    xsa = xs_ref[...].reshape(SN, _B)
    out_ref[...] = jnp.where(msa > 0.5, xsa, o).reshape(_S, _N, _B)


def _kron(W):
    return jnp.kron(W, jnp.eye(_B, dtype=W.dtype))


def _rep(b):
    return jnp.repeat(b, _B)[None, :]


@jax.jit
def kernel(x, edge_index, mask, adj, emb, params):
    del edge_index
    xs = jnp.transpose(x[..., 0], (1, 2, 0))
    ms = jnp.transpose(mask[..., 0].astype(jnp.float32), (1, 2, 0))
    adjT = adj.T
    embr = jnp.repeat(emb, _B, axis=1)

    def dirw(p):
        return [
            _kron(p['Wd']), _rep(p['bd']),
            _kron(jnp.concatenate([p['Wr'], p['Wz']], axis=1)),
            _rep(jnp.concatenate([p['br'], p['bz']])),
            _kron(p['Wc']), _rep(p['bc']),
            _kron(p['Wro']), _rep(p['bro']),
            _kron(p['Wro2']), _rep(p['bro2']),
        ]

    fw = dirw(params['fwd'])
    bw = dirw(params['bwd'])
    Wm1 = _kron(params['Wm1'])
    bm1 = _rep(params['bm1'])
    Wm2 = _kron(params['Wm2'])
    bm2 = _rep(params['bm2'])

    out = pl.pallas_call(
        _grinet_body,
        out_shape=jax.ShapeDtypeStruct((_S, _N, _B), jnp.float32),
        scratch_shapes=[
            pltpu.VMEM((_S, _N, _B), jnp.float32),
            pltpu.VMEM((_S, _N, _B), jnp.float32),
            pltpu.VMEM((_S, _N, _DH * _B), jnp.float32),
            pltpu.VMEM((_S, _N, _DH * _B), jnp.float32),
        ],
    )(xs, ms, adj, adjT, embr, *fw, *bw, Wm1, bm1, Wm2, bm2)

    return jnp.transpose(out, (2, 0, 1))[..., None]
